# baseline (speedup 1.0000x reference)
import numpy as np

import jax
import jax.numpy as jnp
from jax import lax
from jax.experimental import pallas as pl
from jax.experimental.pallas import tpu as pltpu
from jax.experimental.pallas import tpu_sc as plsc

N = 10000
R = 8
D = 128
E = 320000
NTILE = 16
SLICE = 112
NS = 184
NCH = 8
EP = NTILE * NS * SLICE
NSEG = N * R
NSEG_PAD = 80128
ZROWS = 10016

_f32 = jnp.float32
_i32 = jnp.int32
_bf16 = jnp.bfloat16


def _mesh():
    return plsc.VectorSubcoreMesh(core_axis_name="c", subcore_axis_name="s")


def _weights_body(dstp_hbm, etp_hbm, comp_hbm, w1_hbm, w2_hbm, w3_hbm,
                  dst_v, type_v, seg_v, cval_v, wbuf_v, ones_v, comp_v,
                  zc_v, sem, cnt_s):
    c = lax.axis_index("c")
    t = lax.axis_index("s")

    @pl.when(c == 0)
    def _():
        pltpu.sync_copy(dstp_hbm.at[t], dst_v)
        pltpu.sync_copy(etp_hbm.at[t], type_v)
        pltpu.sync_copy(comp_hbm, comp_v)

        for g in range(SLICE // 16):
            ones_v[pl.ds(g * 16, 16)] = jnp.ones((16,), _f32)

        def _seg(i, carry):
            for g in range(SLICE // 16):
                dv = dst_v[i, pl.ds(g * 16, 16)]
                tv = type_v[i, pl.ds(g * 16, 16)]
                seg_v[i, pl.ds(g * 16, 16)] = dv * R + tv
            return carry
        lax.fori_loop(0, NS, _seg, 0)

        def _zc(k, carry):
            zc_v[pl.ds(k * 16, 16)] = jnp.zeros((16,), _f32)
            return carry
        lax.fori_loop(0, 5008 // 16, _zc, 0)
        pltpu.sync_copy(zc_v, cnt_s.at[pl.ds(t * 5008, 5008)])
        plsc.subcore_barrier()

        def _count(s, carry):
            pltpu.sync_copy(ones_v, cnt_s.at[seg_v.at[s]], add=True)
            return carry
        lax.fori_loop(0, NS, _count, 0)
        plsc.subcore_barrier()

        def _gather(s, carry):
            pltpu.async_copy(cnt_s.at[seg_v.at[s]], cval_v.at[s], sem).wait()
            return carry
        lax.fori_loop(0, NS, _gather, 0)

        def _inv(s, carry):
            for g in range(SLICE // 16):
                cv = cval_v[s, pl.ds(g * 16, 16)]
                cval_v[s, pl.ds(g * 16, 16)] = 1.0 / jnp.maximum(cv, 1.0)
            return carry
        lax.fori_loop(0, NS, _inv, 0)

        for l, w_hbm in enumerate((w1_hbm, w2_hbm, w3_hbm)):
            for b in range(2):
                def _w(s, carry, _l=l, _b=b):
                    for g in range(SLICE // 16):
                        tv = type_v[s, pl.ds(g * 16, 16)]
                        iv = cval_v[s, pl.ds(g * 16, 16)]
                        cw = plsc.load_gather(comp_v, [_l * 18 + tv * 2 + _b])
                        wbuf_v[s, pl.ds(g * 16, 16)] = cw * iv
                    return carry
                lax.fori_loop(0, NS, _w, 0)
                pltpu.sync_copy(wbuf_v, w_hbm.at[b, t])


def _weights_call(dstp, etp, comp_flat):
    wshape = jax.ShapeDtypeStruct((2, NTILE, NS, SLICE), _f32)
    return pl.kernel(
        _weights_body,
        out_type=(wshape, wshape, wshape),
        mesh=_mesh(),
        compiler_params=pltpu.CompilerParams(needs_layout_passes=False),
        scratch_types=[
            pltpu.VMEM((NS, SLICE), _i32),
            pltpu.VMEM((NS, SLICE), _i32),
            pltpu.VMEM((NS, SLICE), _i32),
            pltpu.VMEM((NS, SLICE), _f32),
            pltpu.VMEM((NS, SLICE), _f32),
            pltpu.VMEM((SLICE,), _f32),
            pltpu.VMEM((128,), _f32),
            pltpu.VMEM((5008,), _f32),
            pltpu.SemaphoreType.DMA,
            pltpu.VMEM_SHARED((NSEG_PAD,), _f32),
        ],
    )(dstp, etp, comp_flat)


def _scatter_body(xp_hbm, srcp_hbm, dstp_hbm, w_hbm, z_hbm,
                  src_v, dst_v, w_v, f0, f1, f2, gsem, ssem, z_s):
    c = lax.axis_index("c")
    t = lax.axis_index("s")

    def _zr(i, carry):
        for g in range(8):
            f0[i, pl.ds(g * 16, 16)] = jnp.zeros((16,), _f32)
        return carry
    lax.fori_loop(0, SLICE, _zr, 0)

    def _zz(k, carry):
        pltpu.sync_copy(f0, z_s.at[pl.ds(t * 624 + k * SLICE, SLICE)])
        return carry
    lax.fori_loop(0, 5, _zz, 0)
    pltpu.sync_copy(f0.at[pl.ds(0, 64)],
                    z_s.at[pl.ds(t * 624 + 5 * SLICE, 64)])

    @pl.when(t == 15)
    def _():
        pltpu.sync_copy(f0.at[pl.ds(0, 96)], z_s.at[pl.ds(9920, 96)])

    plsc.subcore_barrier()

    def _scale(buf, s):
        def body(e, carry):
            wv = plsc.load_gather(w_v, [jnp.full((16,), s, _i32),
                                        jnp.full((16,), e, _i32)])
            for g in range(8):
                buf[e, pl.ds(g * 16, 16)] = buf[e, pl.ds(g * 16, 16)] * wv
            return carry
        lax.fori_loop(0, SLICE, body, 0)

    def _chunk(k, carry):
        pltpu.sync_copy(srcp_hbm.at[t, pl.ds(k * NCH, NCH)], src_v)
        pltpu.sync_copy(dstp_hbm.at[t, pl.ds(k * NCH, NCH)], dst_v)
        pltpu.sync_copy(w_hbm.at[c, t, pl.ds(k * NCH, NCH)], w_v)

        rb = (f0, f1, f2)
        gd = [None] * NCH
        sc = [None, None, None]
        gd[0] = pltpu.async_copy(xp_hbm.at[src_v.at[0]], f0, gsem)
        gd[1] = pltpu.async_copy(xp_hbm.at[src_v.at[1]], f1, gsem)
        for s in range(NCH):
            cur = rb[s % 3]
            if s + 2 < NCH:
                if sc[(s + 2) % 3] is not None:
                    sc[(s + 2) % 3].wait()
                gd[s + 2] = pltpu.async_copy(xp_hbm.at[src_v.at[s + 2]],
                                             rb[(s + 2) % 3], gsem)
            gd[s].wait()
            _scale(cur, s)
            sc[s % 3] = pltpu.async_copy(cur, z_s.at[dst_v.at[s]], ssem,
                                         add=True)
        for d in sc:
            if d is not None:
                d.wait()
        return carry
    lax.fori_loop(0, NS // NCH, _chunk, 0)
    plsc.subcore_barrier()

    @pl.when(t < 15)
    def _():
        def _dump(k, carry):
            pltpu.sync_copy(z_s.at[pl.ds(t * 624 + k * SLICE, SLICE)],
                            z_hbm.at[c, pl.ds(t * 624 + k * SLICE, SLICE)])
            return carry
        lax.fori_loop(0, 5, _dump, 0)
        pltpu.sync_copy(z_s.at[pl.ds(t * 624 + 5 * SLICE, 64)],
                        z_hbm.at[c, pl.ds(t * 624 + 5 * SLICE, 64)])

    @pl.when(t == 15)
    def _():
        def _dump(k, carry):
            pltpu.sync_copy(z_s.at[pl.ds(9360 + k * SLICE, SLICE)],
                            z_hbm.at[c, pl.ds(9360 + k * SLICE, SLICE)])
            return carry
        lax.fori_loop(0, 5, _dump, 0)
        pltpu.sync_copy(z_s.at[pl.ds(9360 + 5 * SLICE, 80)],
                        z_hbm.at[c, pl.ds(9360 + 5 * SLICE, 80)])


def _scatter_call(xp, srcp, dstp, w):
    return pl.kernel(
        _scatter_body,
        out_type=jax.ShapeDtypeStruct((2, N, D), _f32),
        mesh=_mesh(),
        compiler_params=pltpu.CompilerParams(needs_layout_passes=False),
        scratch_types=[
            pltpu.VMEM((NCH, SLICE), _i32),
            pltpu.VMEM((NCH, SLICE), _i32),
            pltpu.VMEM((NCH, SLICE), _f32),
            pltpu.VMEM((SLICE, D), _f32),
            pltpu.VMEM((SLICE, D), _f32),
            pltpu.VMEM((SLICE, D), _f32),
            pltpu.SemaphoreType.DMA,
            pltpu.SemaphoreType.DMA,
            pltpu.VMEM_SHARED((ZROWS, D), _f32),
        ],
    )(xp, srcp, dstp, w)


def _dense_body(z0_ref, z1_ref, x_ref, basis_ref, root_ref, bias_ref,
                gam_ref, bet_ref, o_ref):
    h = jnp.dot(z0_ref[...], basis_ref[0], preferred_element_type=_f32)
    h = h + jnp.dot(z1_ref[...], basis_ref[1], preferred_element_type=_f32)
    h = h + jnp.dot(x_ref[...], root_ref[...], preferred_element_type=_f32)
    h = h + bias_ref[0]
    mu = jnp.mean(h, axis=-1, keepdims=True)
    d = h - mu
    var = jnp.mean(d * d, axis=-1, keepdims=True)
    y = d * lax.rsqrt(var + 1e-5) * gam_ref[0] + bet_ref[0]
    o_ref[...] = jnp.maximum(y, 0.0)


def _dense3_body(z0_ref, z1_ref, x_ref, basis_ref, root_ref, bias_ref,
                 x0_ref, o_ref):
    h = jnp.dot(z0_ref[...], basis_ref[0], preferred_element_type=_f32)
    h = h + jnp.dot(z1_ref[...], basis_ref[1], preferred_element_type=_f32)
    h = h + jnp.dot(x_ref[...], root_ref[...], preferred_element_type=_f32)
    o_ref[...] = h + bias_ref[0] + x0_ref[...]


_ROWB = 1000


def _row_spec():
    return pl.BlockSpec((_ROWB, D), lambda i: (i, 0))


def _full_spec(shape):
    nd = len(shape)
    return pl.BlockSpec(shape, lambda i: (0,) * nd)


def _dense_call(z0, z1, x, basis, root, bias, gam, bet):
    return pl.pallas_call(
        _dense_body,
        grid=(N // _ROWB,),
        in_specs=[_row_spec(), _row_spec(), _row_spec(),
                  _full_spec((2, D, D)), _full_spec((D, D)),
                  _full_spec((1, D)), _full_spec((1, D)), _full_spec((1, D))],
        out_specs=_row_spec(),
        out_shape=jax.ShapeDtypeStruct((N, D), _f32),
    )(z0, z1, x, basis, root, bias.reshape(1, D), gam.reshape(1, D),
      bet.reshape(1, D))


def _dense3_call(z0, z1, x, basis, root, bias, x0):
    return pl.pallas_call(
        _dense3_body,
        grid=(N // _ROWB,),
        in_specs=[_row_spec(), _row_spec(), _row_spec(),
                  _full_spec((2, D, D)), _full_spec((D, D)),
                  _full_spec((1, D)), _row_spec()],
        out_specs=_row_spec(),
        out_shape=jax.ShapeDtypeStruct((N, D), _f32),
    )(z0, z1, x, basis, root, bias.reshape(1, D), x0)


def kernel(node_ids, edge_index, edge_type, emb,
           basis1, comp1, root1, bias1,
           basis2, comp2, root2, bias2,
           basis3, comp3, root3, bias3,
           ln1_gamma, ln1_beta, ln2_gamma, ln2_beta):
    x = jnp.take(emb, node_ids, axis=0)

    pad = EP - E
    src = edge_index[0]
    dst = edge_index[1]
    srcp = jnp.concatenate([src, jnp.zeros((pad,), _i32)]).reshape(
        NTILE, NS, SLICE)
    dstp = jnp.concatenate([dst, jnp.full((pad,), N, _i32)]).reshape(
        NTILE, NS, SLICE)
    etp = jnp.concatenate([edge_type, jnp.full((pad,), R, _i32)]).reshape(
        NTILE, NS, SLICE)

    comp_flat = jnp.zeros((128,), _f32)
    for l, comp in enumerate((comp1, comp2, comp3)):
        comp_flat = comp_flat.at[l * 18:l * 18 + 16].set(comp.reshape(16))

    w1, w2, w3 = _weights_call(dstp, etp, comp_flat)

    z = _scatter_call(x, srcp, dstp, w1)
    h1 = _dense_call(z[0], z[1], x, basis1, root1, bias1, ln1_gamma, ln1_beta)
    z = _scatter_call(h1, srcp, dstp, w2)
    h2 = _dense_call(z[0], z[1], h1, basis2, root2, bias2, ln2_gamma, ln2_beta)
    z = _scatter_call(h2, srcp, dstp, w3)
    return _dense3_call(z[0], z[1], h2, basis3, root3, bias3, x)


# speedup vs baseline: 1.2260x; 1.2260x over previous
import jax
import jax.numpy as jnp
from jax import lax
from jax.experimental import pallas as pl
from jax.experimental.pallas import tpu as pltpu
from jax.experimental.pallas import tpu_sc as plsc

N = 10000
R = 8
D = 128
E = 320000
NTILE = 16
SLICE = 128
NS = 160
NCH = 16
NCHUNK = NS // NCH
EP = NTILE * NS * SLICE
NSEG = N * R
NSEG_PAD = 80128
ZROWS = 10048

_f32 = jnp.float32
_i32 = jnp.int32


def _mesh():
    return plsc.VectorSubcoreMesh(core_axis_name="c", subcore_axis_name="s")


def _weights_body(srcp_hbm, dstp_hbm, etp_hbm, comp_hbm,
                  m1_hbm, m2_hbm, m3_hbm,
                  dst_v, type_v, seg_v, cval_v, wbuf_v, ones_v, comp_v,
                  zc_v, sem, cnt_s):
    c = lax.axis_index("c")
    t = lax.axis_index("s")

    @pl.when(c == 0)
    def _():
        pltpu.sync_copy(dstp_hbm.at[t], dst_v)
        pltpu.sync_copy(etp_hbm.at[t], type_v)
        pltpu.sync_copy(comp_hbm, comp_v)

        for g in range(8):
            ones_v[pl.ds(g * 16, 16)] = jnp.ones((16,), _f32)

        def _seg(i, carry):
            for g in range(8):
                dv = dst_v[i, pl.ds(g * 16, 16)]
                tv = type_v[i, pl.ds(g * 16, 16)]
                seg_v[i, pl.ds(g * 16, 16)] = dv * R + tv
            return carry
        lax.fori_loop(0, NS, _seg, 0)

        def _zc(k, carry):
            zc_v[pl.ds(k * 16, 16)] = jnp.zeros((16,), _f32)
            return carry
        lax.fori_loop(0, 5008 // 16, _zc, 0)
        pltpu.sync_copy(zc_v, cnt_s.at[pl.ds(t * 5008, 5008)])
        plsc.subcore_barrier()

        def _count(s, carry):
            pltpu.sync_copy(ones_v, cnt_s.at[seg_v.at[s]], add=True)
            return carry
        lax.fori_loop(0, NS, _count, 0)
        plsc.subcore_barrier()

        def _gather(s, carry):
            pltpu.async_copy(cnt_s.at[seg_v.at[s]], cval_v.at[s], sem).wait()
            return carry
        lax.fori_loop(0, NS, _gather, 0)

        def _inv(s, carry):
            for g in range(8):
                cv = cval_v[s, pl.ds(g * 16, 16)]
                cval_v[s, pl.ds(g * 16, 16)] = 1.0 / jnp.maximum(cv, 1.0)
            return carry
        lax.fori_loop(0, NS, _inv, 0)

        for l, m_hbm in enumerate((m1_hbm, m2_hbm, m3_hbm)):
            pltpu.sync_copy(dst_v, m_hbm.at[t, :, 1])
            for b in range(2):
                def _w(s, carry, _l=l, _b=b):
                    for g in range(8):
                        tv = type_v[s, pl.ds(g * 16, 16)]
                        iv = cval_v[s, pl.ds(g * 16, 16)]
                        cw = plsc.load_gather(comp_v, [_l * 18 + tv * 2 + _b])
                        wbuf_v[s, pl.ds(g * 16, 16)] = plsc.bitcast(
                            cw * iv, _i32)
                    return carry
                lax.fori_loop(0, NS, _w, 0)
                pltpu.sync_copy(wbuf_v, m_hbm.at[t, :, 2 + b])

        pltpu.sync_copy(srcp_hbm.at[t], seg_v)
        for m_hbm in (m1_hbm, m2_hbm, m3_hbm):
            pltpu.sync_copy(seg_v, m_hbm.at[t, :, 0])


def _weights_call(srcp, dstp, etp, comp_flat):
    mshape = jax.ShapeDtypeStruct((NTILE, NS, 4, SLICE), _i32)
    return pl.kernel(
        _weights_body,
        out_type=(mshape, mshape, mshape),
        mesh=_mesh(),
        compiler_params=pltpu.CompilerParams(needs_layout_passes=False),
        scratch_types=[
            pltpu.VMEM((NS, SLICE), _i32),
            pltpu.VMEM((NS, SLICE), _i32),
            pltpu.VMEM((NS, SLICE), _i32),
            pltpu.VMEM((NS, SLICE), _f32),
            pltpu.VMEM((NS, SLICE), _i32),
            pltpu.VMEM((SLICE,), _f32),
            pltpu.VMEM((128,), _f32),
            pltpu.VMEM((5008,), _f32),
            pltpu.SemaphoreType.DMA,
            pltpu.VMEM_SHARED((NSEG_PAD,), _f32),
        ],
    )(srcp, dstp, etp, comp_flat)


def _scatter_body(x_hbm, meta_hbm, z_hbm,
                  m0, m1, r0, r1, gsem, ssem, msem, z_s):
    c = lax.axis_index("c")
    t = lax.axis_index("s")
    cvec = jnp.full((16,), 2 + c, _i32)

    def _zr(i, carry):
        for g in range(8):
            r0[i, pl.ds(g * 16, 16)] = jnp.zeros((16,), _f32)
        return carry
    lax.fori_loop(0, SLICE, _zr, 0)

    @pl.when(t < 15)
    def _():
        def _zz(k, carry):
            pltpu.sync_copy(r0, z_s.at[pl.ds(t * 624 + k * 128, 128)])
            return carry
        lax.fori_loop(0, 4, _zz, 0)
        pltpu.sync_copy(r0.at[pl.ds(0, 112)],
                        z_s.at[pl.ds(t * 624 + 512, 112)])

    @pl.when(t == 15)
    def _():
        def _zz(k, carry):
            pltpu.sync_copy(r0, z_s.at[pl.ds(9360 + k * 128, 128)])
            return carry
        lax.fori_loop(0, 5, _zz, 0)
        pltpu.sync_copy(r0.at[pl.ds(0, 48)], z_s.at[pl.ds(10000, 48)])

    plsc.subcore_barrier()

    def _scale(mv, buf, s):
        svec = jnp.full((16,), s, _i32)

        def body(e, evec):
            wv = plsc.bitcast(plsc.load_gather(mv, [svec, cvec, evec]), _f32)
            for g in range(8):
                buf[e, pl.ds(g * 16, 16)] = buf[e, pl.ds(g * 16, 16)] * wv
            return evec + 1
        lax.fori_loop(0, SLICE, body, jnp.zeros((16,), _i32))

    def _run_chunk(mv):
        rb = (r0, r1)
        gd = pltpu.async_copy(x_hbm.at[mv.at[0, 0]], r0, gsem)
        sc = [None, None]
        for s in range(NCH):
            cur = rb[s % 2]
            if s + 1 < NCH:
                gd_next = pltpu.async_copy(x_hbm.at[mv.at[s + 1, 0]],
                                           rb[(s + 1) % 2], gsem)
            gd.wait()
            if sc[s % 2] is not None:
                sc[s % 2].wait()
            _scale(mv, cur, s)
            sc[s % 2] = pltpu.async_copy(cur, z_s.at[mv.at[s, 1]], ssem,
                                         add=True)
            if s + 1 < NCH:
                gd = gd_next
        sc[0].wait()
        sc[1].wait()

    pltpu.sync_copy(meta_hbm.at[t, pl.ds(0, NCH)], m0)

    def _pair(k2, carry):
        k = 2 * k2

        @pl.when(k2 > 0)
        def _():
            pltpu.make_async_copy(meta_hbm.at[t, pl.ds(k * NCH, NCH)],
                                  m0, msem).wait()
        pltpu.async_copy(meta_hbm.at[t, pl.ds((k + 1) * NCH, NCH)], m1, msem)
        _run_chunk(m0)
        pltpu.make_async_copy(meta_hbm.at[t, pl.ds((k + 1) * NCH, NCH)],
                              m1, msem).wait()

        @pl.when(k2 < NCHUNK // 2 - 1)
        def _():
            pltpu.async_copy(meta_hbm.at[t, pl.ds((k + 2) * NCH, NCH)],
                             m0, msem)
        _run_chunk(m1)
        return carry
    lax.fori_loop(0, NCHUNK // 2, _pair, 0)
    plsc.subcore_barrier()

    @pl.when(t < 15)
    def _():
        def _dump(k, carry):
            pltpu.sync_copy(z_s.at[pl.ds(t * 624 + k * 128, 128)],
                            z_hbm.at[c, pl.ds(t * 624 + k * 128, 128)])
            return carry
        lax.fori_loop(0, 4, _dump, 0)
        pltpu.sync_copy(z_s.at[pl.ds(t * 624 + 512, 112)],
                        z_hbm.at[c, pl.ds(t * 624 + 512, 112)])

    @pl.when(t == 15)
    def _():
        def _dump(k, carry):
            pltpu.sync_copy(z_s.at[pl.ds(9360 + k * 128, 128)],
                            z_hbm.at[c, pl.ds(9360 + k * 128, 128)])
            return carry
        lax.fori_loop(0, 5, _dump, 0)


def _scatter_call(x, meta):
    return pl.kernel(
        _scatter_body,
        out_type=jax.ShapeDtypeStruct((2, N, D), _f32),
        mesh=_mesh(),
        compiler_params=pltpu.CompilerParams(needs_layout_passes=False),
        scratch_types=[
            pltpu.VMEM((NCH, 4, SLICE), _i32),
            pltpu.VMEM((NCH, 4, SLICE), _i32),
            pltpu.VMEM((SLICE, D), _f32),
            pltpu.VMEM((SLICE, D), _f32),
            pltpu.SemaphoreType.DMA,
            pltpu.SemaphoreType.DMA,
            pltpu.SemaphoreType.DMA,
            pltpu.VMEM_SHARED((ZROWS, D), _f32),
        ],
    )(x, meta)


def _dense_body(z0_ref, z1_ref, x_ref, basis_ref, root_ref, bias_ref,
                gam_ref, bet_ref, o_ref):
    h = jnp.dot(z0_ref[...], basis_ref[0], preferred_element_type=_f32)
    h = h + jnp.dot(z1_ref[...], basis_ref[1], preferred_element_type=_f32)
    h = h + jnp.dot(x_ref[...], root_ref[...], preferred_element_type=_f32)
    h = h + bias_ref[0]
    mu = jnp.mean(h, axis=-1, keepdims=True)
    d = h - mu
    var = jnp.mean(d * d, axis=-1, keepdims=True)
    y = d * lax.rsqrt(var + 1e-5) * gam_ref[0] + bet_ref[0]
    o_ref[...] = jnp.maximum(y, 0.0)


def _dense3_body(z0_ref, z1_ref, x_ref, basis_ref, root_ref, bias_ref,
                 x0_ref, o_ref):
    h = jnp.dot(z0_ref[...], basis_ref[0], preferred_element_type=_f32)
    h = h + jnp.dot(z1_ref[...], basis_ref[1], preferred_element_type=_f32)
    h = h + jnp.dot(x_ref[...], root_ref[...], preferred_element_type=_f32)
    o_ref[...] = h + bias_ref[0] + x0_ref[...]


_ROWB = 1000


def _row_spec():
    return pl.BlockSpec((_ROWB, D), lambda i: (i, 0))


def _full_spec(shape):
    nd = len(shape)
    return pl.BlockSpec(shape, lambda i: (0,) * nd)


def _dense_call(z0, z1, x, basis, root, bias, gam, bet):
    return pl.pallas_call(
        _dense_body,
        grid=(N // _ROWB,),
        in_specs=[_row_spec(), _row_spec(), _row_spec(),
                  _full_spec((2, D, D)), _full_spec((D, D)),
                  _full_spec((1, D)), _full_spec((1, D)), _full_spec((1, D))],
        out_specs=_row_spec(),
        out_shape=jax.ShapeDtypeStruct((N, D), _f32),
    )(z0, z1, x, basis, root, bias.reshape(1, D), gam.reshape(1, D),
      bet.reshape(1, D))


def _dense3_call(z0, z1, x, basis, root, bias, x0):
    return pl.pallas_call(
        _dense3_body,
        grid=(N // _ROWB,),
        in_specs=[_row_spec(), _row_spec(), _row_spec(),
                  _full_spec((2, D, D)), _full_spec((D, D)),
                  _full_spec((1, D)), _row_spec()],
        out_specs=_row_spec(),
        out_shape=jax.ShapeDtypeStruct((N, D), _f32),
    )(z0, z1, x, basis, root, bias.reshape(1, D), x0)


def kernel(node_ids, edge_index, edge_type, emb,
           basis1, comp1, root1, bias1,
           basis2, comp2, root2, bias2,
           basis3, comp3, root3, bias3,
           ln1_gamma, ln1_beta, ln2_gamma, ln2_beta):
    x = jnp.take(emb, node_ids, axis=0)

    pad = EP - E
    src = edge_index[0]
    dst = edge_index[1]
    srcp = jnp.concatenate([src, jnp.zeros((pad,), _i32)]).reshape(
        NTILE, NS, SLICE)
    dstp = jnp.concatenate([dst, jnp.full((pad,), N, _i32)]).reshape(
        NTILE, NS, SLICE)
    etp = jnp.concatenate([edge_type, jnp.full((pad,), R, _i32)]).reshape(
        NTILE, NS, SLICE)

    comp_flat = jnp.zeros((128,), _f32)
    for l, comp in enumerate((comp1, comp2, comp3)):
        comp_flat = comp_flat.at[l * 18:l * 18 + 16].set(comp.reshape(16))

    meta1, meta2, meta3 = _weights_call(srcp, dstp, etp, comp_flat)

    z = _scatter_call(x, meta1)
    h1 = _dense_call(z[0], z[1], x, basis1, root1, bias1, ln1_gamma, ln1_beta)
    z = _scatter_call(h1, meta2)
    h2 = _dense_call(z[0], z[1], h1, basis2, root2, bias2, ln2_gamma, ln2_beta)
    z = _scatter_call(h2, meta3)
    return _dense3_call(z[0], z[1], h2, basis3, root3, bias3, x)
